# baseline (speedup 1.0000x reference)
import functools

import jax
import jax.numpy as jnp
from jax import lax
from jax.experimental import pallas as pl
from jax.experimental.pallas import tpu as pltpu
from jax.experimental.pallas import tpu_sc as plsc

MARGIN = 0.2
NEG = float("-inf")


def _vmat_body(x_ref, xt_ref, labc_ref, labr_ref, v_ref, k_ref):
    x = x_ref[...]
    xt = xt_ref[...]
    g = jnp.dot(x, xt, preferred_element_type=jnp.float32)
    sqc = jnp.sum(x * x, axis=1, keepdims=True)
    sqr = jnp.sum(xt * xt, axis=0, keepdims=True)
    d2 = jnp.maximum(sqc + sqr - 2.0 * g, 0.0)
    dist = jnp.sqrt(jnp.maximum(d2, 1e-12))
    match = labc_ref[...] == labr_ref[...]
    mf = match.astype(jnp.float32)
    v_ref[...] = -dist + MARGIN * (1.0 - mf)
    k_ref[...] = jnp.sum(mf, axis=1, keepdims=True).astype(jnp.int32)


def _make_sc_loss(n, nc, ns):
    nw = nc * ns
    rpt = n // nw
    nchunk = n // 16
    ngrp = nchunk // 16
    lanes = 16
    mesh = plsc.VectorSubcoreMesh(core_axis_name="c", subcore_axis_name="s",
                                  num_cores=nc, num_subcores=ns)

    @functools.partial(
        pl.kernel,
        out_type=jax.ShapeDtypeStruct((nw, lanes), jnp.float32),
        mesh=mesh,
        compiler_params=pltpu.CompilerParams(needs_layout_passes=False),
        scratch_types=[
            pltpu.VMEM((n,), jnp.float32),
            pltpu.VMEM((n,), jnp.float32),
            pltpu.VMEM((nchunk,), jnp.float32),
            pltpu.VMEM((rpt,), jnp.int32),
            pltpu.VMEM((rpt,), jnp.int32),
            pltpu.VMEM((lanes,), jnp.float32),
            pltpu.SemaphoreType.DMA,
            pltpu.SemaphoreType.DMA,
        ],
    )
    def sc_loss(v_hbm, kv_hbm, st_hbm, out_hbm,
                vrow_a, vrow_b, cmax, kvloc, stloc, accbuf, sem_a, sem_b):
        wid = lax.axis_index("s") * nc + lax.axis_index("c")
        base = wid * rpt
        iota = lax.broadcasted_iota(jnp.int32, (lanes,), 0)
        negv = jnp.full((lanes,), NEG, jnp.float32)
        posv = jnp.full((lanes,), float("inf"), jnp.float32)
        lf_v = jnp.full((lanes,), float(n), jnp.float32)

        pltpu.sync_copy(kv_hbm.at[pl.ds(base, rpt)], kvloc)
        pltpu.sync_copy(st_hbm.at[pl.ds(base, rpt)], stloc)

        def process_row(vrow, i, accv):
            isp = jnp.full((lanes,), i, jnp.int32)
            k = jnp.max(plsc.load_gather(kvloc, [isp]))
            st = jnp.max(plsc.load_gather(stloc, [isp]))
            kv16 = jnp.full((lanes,), k, jnp.int32)
            kf_v = kv16.astype(jnp.float32)

            ng = 8

            def grp_fn(g8, a2):
                t0 = g8 * ng
                mts = []
                for j in range(ng):
                    tj = t0 + j
                    idx = jnp.minimum(st + tj, n - 1)
                    mtj = plsc.load_gather(vrow, [jnp.full((lanes,), idx,
                                                           jnp.int32)])
                    mts.append(jnp.where(jnp.full((lanes,), tj, jnp.int32)
                                         < kv16, mtj, posv))
                zeros = tuple(jnp.zeros((lanes,), jnp.int32)
                              for _ in range(ng))

                @plsc.parallel_loop(0, nchunk // 8, step=1, unroll=2,
                                    carry=zeros)
                def cnt_loop(cg, cvs8):
                    bb = cg * (8 * lanes)
                    out = list(cvs8)
                    for jj in range(8):
                        vc = vrow[pl.ds(bb + jj * lanes, lanes)]
                        for j in range(ng):
                            out[j] = out[j] + (vc > mts[j]).astype(jnp.int32)
                    return tuple(out)

                cvs = cnt_loop
                for j in range(ng):
                    r_v = jnp.full((lanes,), jnp.sum(cvs[j]) + 1,
                                   jnp.int32).astype(jnp.float32)
                    wfp = 0.5 + 0.5 * (kf_v - r_v + 1.0) / kf_v
                    wfn = 0.5 + 0.5 * (r_v - kf_v) / (lf_v - kf_v)
                    w = jnp.where(r_v <= kf_v, wfp, wfn)
                    act = jnp.full((lanes,), t0 + j, jnp.int32) < kv16
                    a2 = a2 + jnp.where(act, w * mts[j],
                                        jnp.zeros((lanes,), jnp.float32))
                return a2

            ngroups = (k + ng - 1) // ng
            acc2 = lax.fori_loop(0, ngroups, grp_fn,
                                 jnp.zeros((lanes,), jnp.float32))

            iota16s = iota * lanes

            def bld_fn(cg, _):
                mx = negv
                for off in range(lanes):
                    gv = plsc.load_gather(vrow, [cg * 256 + iota16s + off])
                    mx = jnp.maximum(mx, gv)
                cmax[pl.ds(cg * lanes, lanes)] = mx
                return 0

            lax.fori_loop(0, ngrp, bld_fn, 0)

            validg = iota < ngrp
            cm2 = negv
            for off in range(lanes):
                gv = plsc.load_gather(cmax,
                                      [jnp.where(validg, iota16s + off, 0)])
                cm2 = jnp.maximum(cm2, jnp.where(validg, gv, negv))

            def ext_fn(e, carry):
                a1, cm2 = carry
                gm_v = jnp.full((lanes,), jnp.max(cm2), jnp.float32)
                g = jnp.min(jnp.where(cm2 == gm_v, iota, lanes))
                cgv = plsc.load_gather(cmax, [g * lanes + iota])
                cl = jnp.min(jnp.where(cgv == gm_v, iota, lanes))
                c = g * lanes + cl
                vc = plsc.load_gather(vrow, [c * lanes + iota])
                lane = jnp.min(jnp.where(vc == gm_v, iota, lanes))
                r_v = jnp.full((lanes,), e + 1, jnp.int32).astype(jnp.float32)
                w1 = 0.5 + 0.5 * (kf_v - r_v + 1.0) / kf_v
                a1 = a1 + w1 * gm_v
                plsc.store_scatter(vrow, [c * lanes + iota], negv,
                                   mask=iota == lane)
                vc2 = jnp.where(iota == lane, negv, vc)
                cmx_v = jnp.full((lanes,), jnp.max(vc2), jnp.float32)
                plsc.store_scatter(cmax, [g * lanes + iota], cmx_v,
                                   mask=iota == cl)
                cgv2 = jnp.where(iota == cl, cmx_v, cgv)
                gmx_v = jnp.full((lanes,), jnp.max(cgv2), jnp.float32)
                g_v = jnp.full((lanes,), g, jnp.int32)
                cm2 = jnp.where(iota == g_v, gmx_v, cm2)
                return a1, cm2

            acc1, _ = lax.fori_loop(0, k, ext_fn,
                                    (jnp.zeros((lanes,), jnp.float32), cm2))
            return accv + acc1 - acc2

        pltpu.async_copy(v_hbm.at[base], vrow_a, sem_a)
        pltpu.async_copy(v_hbm.at[base + 1], vrow_b, sem_b)

        def pair_fn(p, accv):
            i0 = 2 * p
            pltpu.make_async_copy(v_hbm.at[base], vrow_a, sem_a).wait()
            accv = process_row(vrow_a, i0, accv)

            @pl.when(i0 + 2 < rpt)
            def _():
                pltpu.async_copy(v_hbm.at[base + i0 + 2], vrow_a, sem_a)

            pltpu.make_async_copy(v_hbm.at[base], vrow_b, sem_b).wait()
            accv = process_row(vrow_b, i0 + 1, accv)

            @pl.when(i0 + 3 < rpt)
            def _():
                pltpu.async_copy(v_hbm.at[base + i0 + 3], vrow_b, sem_b)

            return accv

        accv = lax.fori_loop(0, rpt // 2, pair_fn,
                             jnp.zeros((lanes,), jnp.float32))
        accbuf[...] = accv
        pltpu.sync_copy(accbuf, out_hbm.at[wid])

    return sc_loss


@jax.jit
def kernel(batch_reprs, batch_labels):
    x = batch_reprs.astype(jnp.float32)
    n, d = x.shape
    labels = batch_labels.reshape(-1)
    counts = jnp.zeros((512,), jnp.int32).at[labels].add(1)
    k_row = counts[labels]
    key = k_row * 4096 + labels
    order = jnp.argsort(key)
    skey = key[order]
    start = jnp.searchsorted(skey, skey, side="left").astype(jnp.int32)
    x = x[order]
    labf = labels[order].astype(jnp.float32)
    lab_col = labf.reshape(n, 1)
    lab_row = labf.reshape(1, n)
    xt = x.T

    rows = 512 if n % 512 == 0 else n
    nblk = n // rows

    v, ks = pl.pallas_call(
        _vmat_body,
        grid=(nblk,),
        in_specs=[
            pl.BlockSpec((rows, d), lambda i: (i, 0)),
            pl.BlockSpec((d, n), lambda i: (0, 0)),
            pl.BlockSpec((rows, 1), lambda i: (i, 0)),
            pl.BlockSpec((1, n), lambda i: (0, 0)),
        ],
        out_specs=[
            pl.BlockSpec((rows, n), lambda i: (i, 0)),
            pl.BlockSpec((rows, 1), lambda i: (i, 0)),
        ],
        out_shape=[
            jax.ShapeDtypeStruct((n, n), jnp.float32),
            jax.ShapeDtypeStruct((n, 1), jnp.int32),
        ],
    )(x, xt, lab_col, lab_row)

    try:
        info = plsc.get_sparse_core_info()
        nc, ns = info.num_cores, info.num_subcores
    except Exception:
        nc, ns = 2, 16
    sc_loss = _make_sc_loss(n, nc, ns)
    parts = sc_loss(v, ks.reshape(-1), start)
    return jnp.sum(parts[:, 0])


# speedup vs baseline: 1.8864x; 1.8864x over previous
import functools

import jax
import jax.numpy as jnp
from jax import lax
from jax.experimental import pallas as pl
from jax.experimental.pallas import tpu as pltpu
from jax.experimental.pallas import tpu_sc as plsc

MARGIN = 0.2
NEG = float("-inf")


def _vmat_body(x_ref, xt_ref, labc_ref, labr_ref, v_ref, k_ref):
    x = x_ref[...]
    xt = xt_ref[...]
    g = jnp.dot(x, xt, preferred_element_type=jnp.float32)
    sqc = jnp.sum(x * x, axis=1, keepdims=True)
    sqr = jnp.sum(xt * xt, axis=0, keepdims=True)
    d2 = jnp.maximum(sqc + sqr - 2.0 * g, 0.0)
    dist = jnp.sqrt(jnp.maximum(d2, 1e-12))
    match = labc_ref[...] == labr_ref[...]
    mf = match.astype(jnp.float32)
    v_ref[...] = -dist + MARGIN * (1.0 - mf)
    k_ref[...] = jnp.sum(mf, axis=1, keepdims=True).astype(jnp.int32)


def _make_sc_loss(n, nc, ns):
    nw = nc * ns
    rpt = n // nw
    nchunk = n // 16
    ngrp = nchunk // 16
    lanes = 16
    mesh = plsc.VectorSubcoreMesh(core_axis_name="c", subcore_axis_name="s",
                                  num_cores=nc, num_subcores=ns)

    @functools.partial(
        pl.kernel,
        out_type=jax.ShapeDtypeStruct((nw, lanes), jnp.float32),
        mesh=mesh,
        compiler_params=pltpu.CompilerParams(needs_layout_passes=False),
        scratch_types=[
            pltpu.VMEM((n,), jnp.float32),
            pltpu.VMEM((n,), jnp.float32),
            pltpu.VMEM((nchunk,), jnp.float32),
            pltpu.VMEM((rpt,), jnp.int32),
            pltpu.VMEM((rpt,), jnp.int32),
            pltpu.VMEM((lanes,), jnp.float32),
            pltpu.SemaphoreType.DMA,
            pltpu.SemaphoreType.DMA,
        ],
    )
    def sc_loss(v_hbm, kv_hbm, st_hbm, out_hbm,
                vrow_a, vrow_b, cmax, kvloc, stloc, accbuf, sem_a, sem_b):
        wid = lax.axis_index("s") * nc + lax.axis_index("c")
        base = wid * rpt
        iota = lax.broadcasted_iota(jnp.int32, (lanes,), 0)
        negv = jnp.full((lanes,), NEG, jnp.float32)
        posv = jnp.full((lanes,), float("inf"), jnp.float32)
        lf_v = jnp.full((lanes,), float(n), jnp.float32)

        pltpu.sync_copy(kv_hbm.at[pl.ds(base, rpt)], kvloc)
        pltpu.sync_copy(st_hbm.at[pl.ds(base, rpt)], stloc)

        def process_row(vrow, i, accv):
            isp = jnp.full((lanes,), i, jnp.int32)
            k = jnp.max(plsc.load_gather(kvloc, [isp]))
            st = jnp.max(plsc.load_gather(stloc, [isp]))
            kv16 = jnp.full((lanes,), k, jnp.int32)
            kf_v = kv16.astype(jnp.float32)

            ng = 4

            def grp_fn(g8, a2):
                t0 = g8 * ng
                mts = []
                for j in range(ng):
                    tj = t0 + j
                    idx = jnp.minimum(st + tj, n - 1)
                    mtj = plsc.load_gather(vrow, [jnp.full((lanes,), idx,
                                                           jnp.int32)])
                    mts.append(jnp.where(jnp.full((lanes,), tj, jnp.int32)
                                         < kv16, mtj, posv))
                zeros = tuple(jnp.zeros((lanes,), jnp.int32)
                              for _ in range(ng))

                @plsc.parallel_loop(0, nchunk // 8, step=1, unroll=2,
                                    carry=zeros)
                def cnt_loop(cg, cvs8):
                    bb = cg * (8 * lanes)
                    out = list(cvs8)
                    for jj in range(8):
                        vc = vrow[pl.ds(bb + jj * lanes, lanes)]
                        for j in range(ng):
                            out[j] = out[j] + (vc > mts[j]).astype(jnp.int32)
                    return tuple(out)

                cvs = cnt_loop
                for j in range(ng):
                    r_v = jnp.full((lanes,), jnp.sum(cvs[j]) + 1,
                                   jnp.int32).astype(jnp.float32)
                    wfp = 0.5 + 0.5 * (kf_v - r_v + 1.0) / kf_v
                    wfn = 0.5 + 0.5 * (r_v - kf_v) / (lf_v - kf_v)
                    w = jnp.where(r_v <= kf_v, wfp, wfn)
                    act = jnp.full((lanes,), t0 + j, jnp.int32) < kv16
                    a2 = a2 + jnp.where(act, w * mts[j],
                                        jnp.zeros((lanes,), jnp.float32))
                return a2

            ngroups = (k + ng - 1) // ng
            acc2 = lax.fori_loop(0, ngroups, grp_fn,
                                 jnp.zeros((lanes,), jnp.float32))

            iota16s = iota * lanes

            def bld_fn(cg, _):
                mx = negv
                for off in range(lanes):
                    gv = plsc.load_gather(vrow, [cg * 256 + iota16s + off])
                    mx = jnp.maximum(mx, gv)
                cmax[pl.ds(cg * lanes, lanes)] = mx
                return 0

            lax.fori_loop(0, ngrp, bld_fn, 0)

            validg = iota < ngrp
            cm2 = negv
            for off in range(lanes):
                gv = plsc.load_gather(cmax,
                                      [jnp.where(validg, iota16s + off, 0)])
                cm2 = jnp.maximum(cm2, jnp.where(validg, gv, negv))

            def ext_fn(e, carry):
                a1, cm2 = carry
                gm_v = jnp.full((lanes,), jnp.max(cm2), jnp.float32)
                g = jnp.min(jnp.where(cm2 == gm_v, iota, lanes))
                cgv = plsc.load_gather(cmax, [g * lanes + iota])
                cl = jnp.min(jnp.where(cgv == gm_v, iota, lanes))
                c = g * lanes + cl
                vc = plsc.load_gather(vrow, [c * lanes + iota])
                lane = jnp.min(jnp.where(vc == gm_v, iota, lanes))
                r_v = jnp.full((lanes,), e + 1, jnp.int32).astype(jnp.float32)
                w1 = 0.5 + 0.5 * (kf_v - r_v + 1.0) / kf_v
                a1 = a1 + w1 * gm_v
                plsc.store_scatter(vrow, [c * lanes + iota], negv,
                                   mask=iota == lane)
                vc2 = jnp.where(iota == lane, negv, vc)
                cmx_v = jnp.full((lanes,), jnp.max(vc2), jnp.float32)
                plsc.store_scatter(cmax, [g * lanes + iota], cmx_v,
                                   mask=iota == cl)
                cgv2 = jnp.where(iota == cl, cmx_v, cgv)
                gmx_v = jnp.full((lanes,), jnp.max(cgv2), jnp.float32)
                g_v = jnp.full((lanes,), g, jnp.int32)
                cm2 = jnp.where(iota == g_v, gmx_v, cm2)
                return a1, cm2

            acc1, _ = lax.fori_loop(0, k, ext_fn,
                                    (jnp.zeros((lanes,), jnp.float32), cm2))
            return accv + acc1 - acc2

        pltpu.async_copy(v_hbm.at[base], vrow_a, sem_a)
        pltpu.async_copy(v_hbm.at[base + 1], vrow_b, sem_b)

        def pair_fn(p, accv):
            i0 = 2 * p
            pltpu.make_async_copy(v_hbm.at[base], vrow_a, sem_a).wait()
            accv = process_row(vrow_a, i0, accv)

            @pl.when(i0 + 2 < rpt)
            def _():
                pltpu.async_copy(v_hbm.at[base + i0 + 2], vrow_a, sem_a)

            pltpu.make_async_copy(v_hbm.at[base], vrow_b, sem_b).wait()
            accv = process_row(vrow_b, i0 + 1, accv)

            @pl.when(i0 + 3 < rpt)
            def _():
                pltpu.async_copy(v_hbm.at[base + i0 + 3], vrow_b, sem_b)

            return accv

        accv = lax.fori_loop(0, rpt // 2, pair_fn,
                             jnp.zeros((lanes,), jnp.float32))
        accbuf[...] = accv
        pltpu.sync_copy(accbuf, out_hbm.at[wid])

    return sc_loss


@jax.jit
def kernel(batch_reprs, batch_labels):
    x = batch_reprs.astype(jnp.float32)
    n, d = x.shape
    labels = batch_labels.reshape(-1)
    counts = jnp.zeros((512,), jnp.int32).at[labels].add(1)
    k_row = counts[labels]
    key = k_row * 4096 + labels
    order = jnp.argsort(key)
    skey = key[order]
    start = jnp.searchsorted(skey, skey, side="left").astype(jnp.int32)
    x = x[order]
    labf = labels[order].astype(jnp.float32)
    lab_col = labf.reshape(n, 1)
    lab_row = labf.reshape(1, n)
    xt = x.T

    rows = 512 if n % 512 == 0 else n
    nblk = n // rows

    v, ks = pl.pallas_call(
        _vmat_body,
        grid=(nblk,),
        in_specs=[
            pl.BlockSpec((rows, d), lambda i: (i, 0)),
            pl.BlockSpec((d, n), lambda i: (0, 0)),
            pl.BlockSpec((rows, 1), lambda i: (i, 0)),
            pl.BlockSpec((1, n), lambda i: (0, 0)),
        ],
        out_specs=[
            pl.BlockSpec((rows, n), lambda i: (i, 0)),
            pl.BlockSpec((rows, 1), lambda i: (i, 0)),
        ],
        out_shape=[
            jax.ShapeDtypeStruct((n, n), jnp.float32),
            jax.ShapeDtypeStruct((n, 1), jnp.int32),
        ],
    )(x, xt, lab_col, lab_row)

    try:
        info = plsc.get_sparse_core_info()
        nc, ns = info.num_cores, info.num_subcores
    except Exception:
        nc, ns = 2, 16
    sc_loss = _make_sc_loss(n, nc, ns)
    parts = sc_loss(v, ks.reshape(-1), start)
    return jnp.sum(parts[:, 0])
